# baseline (speedup 1.0000x reference)
import functools

import jax
import jax.numpy as jnp
from jax import lax
from jax.experimental import pallas as pl
from jax.experimental.pallas import tpu as pltpu
from jax.experimental.pallas import tpu_sc as plsc

_DP = 128
_L = 16


@functools.cache
def _make_lookup(BATCH: int, SEQ: int, D: int):
    info = plsc.get_sparse_core_info()
    NC, NS = info.num_cores, info.num_subcores
    NW = NC * NS
    assert BATCH % (NW * _DP) == 0 and D % _L == 0 and SEQ % 2 == 0
    BB = BATCH // NW
    n_idx = BB * SEQ
    mesh = plsc.VectorSubcoreMesh(core_axis_name="c", subcore_axis_name="s")

    @functools.partial(
        pl.kernel,
        out_type=jax.ShapeDtypeStruct((SEQ, D, BATCH), jnp.float32),
        mesh=mesh,
        scratch_types=[
            pltpu.VMEM((n_idx,), jnp.int32),
            pltpu.VMEM((SEQ, _DP), jnp.int32),
            pltpu.VMEM((2, _DP, _DP), jnp.float32),
            pltpu.VMEM((2, D, _DP), jnp.float32),
            pltpu.SemaphoreType.DMA,
            pltpu.SemaphoreType.DMA,
            pltpu.SemaphoreType.DMA,
            pltpu.SemaphoreType.DMA,
        ],
        compiler_params=pltpu.CompilerParams(
            use_tc_tiling_on_sc=True, needs_layout_passes=False
        ),
    )
    def lookup(table_hbm, idx_hbm, out_hbm, idx_v, idx_t, rows_v, trn_v,
               gsem0, gsem1, osem0, osem1):
        gsem = (gsem0, gsem1)
        osem = (osem0, osem1)
        wid = lax.axis_index("s") * NC + lax.axis_index("c")
        b0 = wid * BB
        pltpu.sync_copy(idx_hbm.at[pl.ds(b0 * SEQ, n_idx)], idx_v)

        lanes = lax.iota(jnp.int32, _L)
        stride_s = lanes * SEQ
        row_m = [lanes + m * _L for m in range(_DP // _L)]

        @pl.loop(0, SEQ)
        def _regroup(s):
            for q in range(_DP // _L):
                vec = plsc.load_gather(idx_v, [stride_s + (s + q * _L * SEQ)])
                idx_t[s, pl.ds(q * _L, _L)] = vec

        def fire_gather(s, buf):
            pltpu.async_copy(
                table_hbm.at[idx_t.at[s]],
                rows_v.at[buf],
                gsem[buf],
            )

        def wait_gather(buf):
            pltpu.make_async_copy(
                table_hbm.at[pl.ds(0, _DP)], rows_v.at[buf], gsem[buf]
            ).wait()

        def transpose(buf):
            @pl.loop(0, D, unroll=2)
            def _t(d):
                col = lanes * 0 + d
                for m in range(_DP // _L):
                    vec = plsc.load_gather(rows_v.at[buf], [row_m[m], col])
                    trn_v[buf, d, pl.ds(m * _L, _L)] = vec

        def fire_out(s, buf):
            pltpu.async_copy(
                trn_v.at[buf], out_hbm.at[s, :, pl.ds(b0, _DP)], osem[buf]
            )

        def wait_out(buf):
            pltpu.make_async_copy(
                trn_v.at[buf], out_hbm.at[0, :, pl.ds(0, _DP)], osem[buf]
            ).wait()

        fire_gather(0, 0)

        @pl.loop(0, SEQ // 2)
        def _step(g):
            for p in range(2):
                s = 2 * g + p

                if p == 0:
                    fire_gather(s + 1, 1 - p)
                else:
                    @pl.when(g < SEQ // 2 - 1)
                    def _():
                        fire_gather(s + 1, 1 - p)

                wait_gather(p)

                @pl.when(g >= 1)
                def _():
                    wait_out(p)

                transpose(p)
                fire_out(s, p)

        wait_out(0)
        wait_out(1)

    return lookup


def kernel(x, table):
    BATCH, SEQ = x.shape
    V, D = table.shape
    table_p = jnp.pad(table, ((0, 0), (0, _DP - D)))
    out_t = _make_lookup(BATCH, SEQ, D)(table_p, x.reshape(-1))
    return out_t.transpose(2, 0, 1)
